# baseline (speedup 1.0000x reference)
import functools

import jax
import jax.numpy as jnp
from jax import lax
from jax.experimental import pallas as pl
from jax.experimental.pallas import tpu as pltpu
from jax.experimental.pallas import tpu_sc as plsc

N_NODES = 10000
N_EDGES = 320000
D = 128

NP = 10240
NC, NS = 2, 16
NW = NC * NS
NPT = NP // NS

EPWP = 10240
EPAD = NW * EPWP

CHD = 80
NCHD = EPWP // CHD

CHA = 80
NCHA = EPWP // CHA

_MESH = plsc.VectorSubcoreMesh(
    core_axis_name="c", subcore_axis_name="s", num_cores=NC, num_subcores=NS
)


@functools.partial(
    pl.kernel,
    out_type=jax.ShapeDtypeStruct((NC, NP), jnp.float32),
    mesh=_MESH,
    scratch_types=[
        pltpu.VMEM((NCHD, CHD), jnp.int32),
        pltpu.VMEM((NPT,), jnp.float32),
        pltpu.VMEM((CHD,), jnp.float32),
        pltpu.VMEM_SHARED((NP,), jnp.float32),
        pltpu.SemaphoreType.DMA((2,)),
    ],
)
def _deg_kernel(row_hbm, out_hbm, row_v, zbuf, ones_v, hist_sh, hsem):
    c = lax.axis_index("c")
    s = lax.axis_index("s")
    wid = s * NC + c

    @pl.loop(0, NPT // 16)
    def _zero(i):
        zbuf[pl.ds(i * 16, 16)] = jnp.zeros((16,), jnp.float32)

    @pl.loop(0, CHD // 16)
    def _one(i):
        ones_v[pl.ds(i * 16, 16)] = jnp.ones((16,), jnp.float32)

    pltpu.sync_copy(zbuf, hist_sh.at[pl.ds(s * NPT, NPT)])
    pltpu.sync_copy(row_hbm.at[wid], row_v)
    plsc.subcore_barrier()

    def hs(j, b):
        pltpu.async_copy(ones_v, hist_sh.at[row_v.at[j]], hsem.at[b],
                         add=True)

    def hw(j, b):
        pltpu.make_async_copy(ones_v, hist_sh.at[row_v.at[j]],
                              hsem.at[b]).wait()

    hs(0, 0)
    hs(1, 1)

    @pl.loop(1, NCHD // 2)
    def _hist(t):
        for b in (0, 1):
            j = t * 2 + b
            hw(j - 2, b)
            hs(j, b)

    hw(NCHD - 2, 0)
    hw(NCHD - 1, 1)
    plsc.subcore_barrier()
    pltpu.sync_copy(hist_sh.at[pl.ds(s * NPT, NPT)],
                    out_hbm.at[c, pl.ds(s * NPT, NPT)])


@functools.partial(
    pl.kernel,
    out_type=jax.ShapeDtypeStruct((NC, NP, D), jnp.float32),
    mesh=_MESH,
    scratch_types=[
        pltpu.VMEM((EPWP,), jnp.int32),
        pltpu.VMEM((NCHA, CHA), jnp.int32),
        pltpu.VMEM((2, CHA, D), jnp.float32),
        pltpu.VMEM_SHARED((NP, D), jnp.float32),
        pltpu.SemaphoreType.DMA((2,)),
        pltpu.SemaphoreType.DMA((2,)),
    ],
)
def _agg_kernel(y_hbm, col_hbm, row_hbm, out_hbm,
                col_v, row_v, gbuf, s_sh, gsem, ssem):
    c = lax.axis_index("c")
    s = lax.axis_index("s")
    wid = s * NC + c

    @pl.when(c == 0)
    def _init_y():
        pltpu.sync_copy(y_hbm.at[pl.ds(s * NPT, NPT)],
                        s_sh.at[pl.ds(s * NPT, NPT)])

    @pl.when(c != 0)
    def _init_z():
        @pl.loop(0, CHA * (D // 16))
        def _zero(i):
            gbuf[0, i // (D // 16), pl.ds((i % (D // 16)) * 16, 16)] = (
                jnp.zeros((16,), jnp.float32))

        @pl.loop(0, NPT // CHA)
        def _zinit(k):
            pltpu.sync_copy(gbuf.at[0],
                            s_sh.at[pl.ds(s * NPT + k * CHA, CHA)])

    pltpu.sync_copy(col_hbm.at[wid], col_v)
    pltpu.sync_copy(row_hbm.at[wid], row_v)
    plsc.subcore_barrier()

    H = CHA // 2

    def sg(j, b):
        pltpu.async_copy(y_hbm.at[col_v.at[pl.ds(j * CHA, H)]],
                         gbuf.at[b, pl.ds(0, H)], gsem.at[b])
        pltpu.async_copy(y_hbm.at[col_v.at[pl.ds(j * CHA + H, H)]],
                         gbuf.at[b, pl.ds(H, H)], gsem.at[b])

    def wg(j, b):
        pltpu.make_async_copy(y_hbm.at[col_v.at[pl.ds(j * CHA, H)]],
                              gbuf.at[b, pl.ds(0, H)], gsem.at[b]).wait()
        pltpu.make_async_copy(y_hbm.at[col_v.at[pl.ds(j * CHA + H, H)]],
                              gbuf.at[b, pl.ds(H, H)], gsem.at[b]).wait()

    def ss(j, b):
        pltpu.async_copy(gbuf.at[b], s_sh.at[row_v.at[j]],
                         ssem.at[b], add=True)

    def ws(j, b):
        pltpu.make_async_copy(gbuf.at[b], s_sh.at[row_v.at[j]],
                              ssem.at[b]).wait()

    sg(0, 0)
    sg(1, 1)
    wg(0, 0)
    ss(0, 0)

    @pl.loop(1, NCHA // 2)
    def _edges(t):
        for b in (0, 1):
            j = t * 2 + b
            ws(j - 2, b)
            sg(j, b)
            wg(j - 1, 1 - b)
            ss(j - 1, 1 - b)

    wg(NCHA - 1, 1)
    ss(NCHA - 1, 1)
    ws(NCHA - 2, 0)
    ws(NCHA - 1, 1)

    plsc.subcore_barrier()
    pltpu.sync_copy(s_sh.at[pl.ds(s * NPT, NPT)],
                    out_hbm.at[c, pl.ds(s * NPT, NPT)])


def _scale_body(h0_ref, h1_ref, x_ref, ybar_ref, dinv_ref):
    deg = h0_ref[...] + h1_ref[...] + 1.0
    dinv = lax.rsqrt(deg)
    dinv_ref[...] = dinv
    ybar_ref[...] = dinv * x_ref[...]


_scale_call = pl.pallas_call(
    _scale_body,
    grid=(NP // 1024,),
    in_specs=[
        pl.BlockSpec((1024, 1), lambda i: (i, 0)),
        pl.BlockSpec((1024, 1), lambda i: (i, 0)),
        pl.BlockSpec((1024, D), lambda i: (i, 0)),
    ],
    out_specs=[
        pl.BlockSpec((1024, D), lambda i: (i, 0)),
        pl.BlockSpec((1024, 1), lambda i: (i, 0)),
    ],
    out_shape=[
        jax.ShapeDtypeStruct((NP, D), jnp.float32),
        jax.ShapeDtypeStruct((NP, 1), jnp.float32),
    ],
)


def _final_body(s0_ref, s1_ref, dinv_ref, w_ref, out_ref):
    z = (s0_ref[...] + s1_ref[...]) * dinv_ref[...]
    out_ref[...] = lax.dot_general(
        z, w_ref[...], (((1,), (1,)), ((), ())),
        preferred_element_type=jnp.float32)


_final_call = pl.pallas_call(
    _final_body,
    grid=(NP // 1024,),
    in_specs=[
        pl.BlockSpec((1024, D), lambda i: (i, 0)),
        pl.BlockSpec((1024, D), lambda i: (i, 0)),
        pl.BlockSpec((1024, 1), lambda i: (i, 0)),
        pl.BlockSpec((D, D), lambda i: (0, 0)),
    ],
    out_specs=pl.BlockSpec((1024, D), lambda i: (i, 0)),
    out_shape=jax.ShapeDtypeStruct((NP, D), jnp.float32),
)


def kernel(x, edge_index, W):
    ei = edge_index.astype(jnp.int32)
    pad = (jnp.arange(EPAD - N_EDGES, dtype=jnp.int32) % (NP - N_NODES)
           ) + N_NODES
    rowp = jnp.concatenate([ei[0], pad])
    colp = jnp.concatenate([ei[1], pad])
    row_deg = rowp.reshape(NW, NCHD, CHD)
    row_agg = rowp.reshape(NW, NCHA, CHA)
    col_agg = colp.reshape(NW, EPWP)
    x_pad = jnp.pad(x, ((0, NP - N_NODES), (0, 0)))

    hist = _deg_kernel(row_deg)
    ybar, dinv = _scale_call(hist[0].reshape(NP, 1), hist[1].reshape(NP, 1),
                             x_pad)
    spart = _agg_kernel(ybar, col_agg, row_agg)
    outp = _final_call(spart[0], spart[1], dinv, W)
    return outp[:N_NODES]


# speedup vs baseline: 1.0139x; 1.0139x over previous
import functools

import jax
import jax.numpy as jnp
from jax import lax
from jax.experimental import pallas as pl
from jax.experimental.pallas import tpu as pltpu
from jax.experimental.pallas import tpu_sc as plsc

N_NODES = 10000
N_EDGES = 320000
D = 128

NP = 10240
NC, NS = 2, 16
NW = NC * NS
NPT = NP // NS

EPWP = 10240
EPAD = NW * EPWP

CHD = 128
NCHD = EPWP // CHD

CHA = 80
NCHA = EPWP // CHA

_MESH = plsc.VectorSubcoreMesh(
    core_axis_name="c", subcore_axis_name="s", num_cores=NC, num_subcores=NS
)


@functools.partial(
    pl.kernel,
    out_type=jax.ShapeDtypeStruct((NC, NP), jnp.float32),
    mesh=_MESH,
    scratch_types=[
        pltpu.VMEM((NCHD, CHD), jnp.int32),
        pltpu.VMEM((NPT,), jnp.float32),
        pltpu.VMEM((CHD,), jnp.float32),
        pltpu.VMEM_SHARED((NP,), jnp.float32),
        pltpu.SemaphoreType.DMA((2,)),
    ],
)
def _deg_kernel(row_hbm, out_hbm, row_v, zbuf, ones_v, hist_sh, hsem):
    c = lax.axis_index("c")
    s = lax.axis_index("s")
    wid = s * NC + c

    @pl.loop(0, NPT // 16)
    def _zero(i):
        zbuf[pl.ds(i * 16, 16)] = jnp.zeros((16,), jnp.float32)

    @pl.loop(0, CHD // 16)
    def _one(i):
        ones_v[pl.ds(i * 16, 16)] = jnp.ones((16,), jnp.float32)

    pltpu.sync_copy(zbuf, hist_sh.at[pl.ds(s * NPT, NPT)])
    pltpu.sync_copy(row_hbm.at[wid], row_v)
    plsc.subcore_barrier()

    def hs(j, b):
        pltpu.async_copy(ones_v, hist_sh.at[row_v.at[j]], hsem.at[b],
                         add=True)

    def hw(j, b):
        pltpu.make_async_copy(ones_v, hist_sh.at[row_v.at[j]],
                              hsem.at[b]).wait()

    hs(0, 0)
    hs(1, 1)

    @pl.loop(1, NCHD // 2)
    def _hist(t):
        for b in (0, 1):
            j = t * 2 + b
            hw(j - 2, b)
            hs(j, b)

    hw(NCHD - 2, 0)
    hw(NCHD - 1, 1)
    plsc.subcore_barrier()
    pltpu.sync_copy(hist_sh.at[pl.ds(s * NPT, NPT)],
                    out_hbm.at[c, pl.ds(s * NPT, NPT)])


@functools.partial(
    pl.kernel,
    out_type=jax.ShapeDtypeStruct((NC, NP, D), jnp.float32),
    mesh=_MESH,
    scratch_types=[
        pltpu.VMEM((EPWP,), jnp.int32),
        pltpu.VMEM((NCHA, CHA), jnp.int32),
        pltpu.VMEM((2, CHA, D), jnp.float32),
        pltpu.VMEM_SHARED((NP, D), jnp.float32),
        pltpu.SemaphoreType.DMA((2,)),
        pltpu.SemaphoreType.DMA((2,)),
    ],
)
def _agg_kernel(y_hbm, col_hbm, row_hbm, out_hbm,
                col_v, row_v, gbuf, s_sh, gsem, ssem):
    c = lax.axis_index("c")
    s = lax.axis_index("s")
    wid = s * NC + c

    @pl.when(c == 0)
    def _init_y():
        pltpu.sync_copy(y_hbm.at[pl.ds(s * NPT, NPT)],
                        s_sh.at[pl.ds(s * NPT, NPT)])

    @pl.when(c != 0)
    def _init_z():
        @pl.loop(0, CHA * (D // 16))
        def _zero(i):
            gbuf[0, i // (D // 16), pl.ds((i % (D // 16)) * 16, 16)] = (
                jnp.zeros((16,), jnp.float32))

        @pl.loop(0, NPT // CHA)
        def _zinit(k):
            pltpu.sync_copy(gbuf.at[0],
                            s_sh.at[pl.ds(s * NPT + k * CHA, CHA)])

    pltpu.sync_copy(col_hbm.at[wid], col_v)
    pltpu.sync_copy(row_hbm.at[wid], row_v)
    plsc.subcore_barrier()

    def sg(j, b):
        pltpu.async_copy(y_hbm.at[col_v.at[pl.ds(j * CHA, CHA)]],
                         gbuf.at[b], gsem.at[b])

    def wg(j, b):
        pltpu.make_async_copy(y_hbm.at[col_v.at[pl.ds(j * CHA, CHA)]],
                              gbuf.at[b], gsem.at[b]).wait()

    def ss(j, b):
        pltpu.async_copy(gbuf.at[b], s_sh.at[row_v.at[j]],
                         ssem.at[b], add=True)

    def ws(j, b):
        pltpu.make_async_copy(gbuf.at[b], s_sh.at[row_v.at[j]],
                              ssem.at[b]).wait()

    sg(0, 0)
    sg(1, 1)
    wg(0, 0)
    ss(0, 0)

    @pl.loop(1, NCHA // 2)
    def _edges(t):
        for b in (0, 1):
            j = t * 2 + b
            ws(j - 2, b)
            sg(j, b)
            wg(j - 1, 1 - b)
            ss(j - 1, 1 - b)

    wg(NCHA - 1, 1)
    ss(NCHA - 1, 1)
    ws(NCHA - 2, 0)
    ws(NCHA - 1, 1)

    plsc.subcore_barrier()
    pltpu.sync_copy(s_sh.at[pl.ds(s * NPT, NPT)],
                    out_hbm.at[c, pl.ds(s * NPT, NPT)])


def _scale_body(h0_ref, h1_ref, x_ref, ybar_ref, dinv_ref):
    deg = h0_ref[...] + h1_ref[...] + 1.0
    dinv = lax.rsqrt(deg)
    dinv_ref[...] = dinv
    ybar_ref[...] = dinv * x_ref[...]


_scale_call = pl.pallas_call(
    _scale_body,
    grid=(NP // 1024,),
    in_specs=[
        pl.BlockSpec((1024, 1), lambda i: (i, 0)),
        pl.BlockSpec((1024, 1), lambda i: (i, 0)),
        pl.BlockSpec((1024, D), lambda i: (i, 0)),
    ],
    out_specs=[
        pl.BlockSpec((1024, D), lambda i: (i, 0)),
        pl.BlockSpec((1024, 1), lambda i: (i, 0)),
    ],
    out_shape=[
        jax.ShapeDtypeStruct((NP, D), jnp.float32),
        jax.ShapeDtypeStruct((NP, 1), jnp.float32),
    ],
)


def _final_body(s0_ref, s1_ref, dinv_ref, w_ref, out_ref):
    z = (s0_ref[...] + s1_ref[...]) * dinv_ref[...]
    out_ref[...] = lax.dot_general(
        z, w_ref[...], (((1,), (1,)), ((), ())),
        preferred_element_type=jnp.float32)


_final_call = pl.pallas_call(
    _final_body,
    grid=(NP // 1024,),
    in_specs=[
        pl.BlockSpec((1024, D), lambda i: (i, 0)),
        pl.BlockSpec((1024, D), lambda i: (i, 0)),
        pl.BlockSpec((1024, 1), lambda i: (i, 0)),
        pl.BlockSpec((D, D), lambda i: (0, 0)),
    ],
    out_specs=pl.BlockSpec((1024, D), lambda i: (i, 0)),
    out_shape=jax.ShapeDtypeStruct((NP, D), jnp.float32),
)


def kernel(x, edge_index, W):
    ei = edge_index.astype(jnp.int32)
    pad = (jnp.arange(EPAD - N_EDGES, dtype=jnp.int32) % (NP - N_NODES)
           ) + N_NODES
    rowp = jnp.concatenate([ei[0], pad])
    colp = jnp.concatenate([ei[1], pad])
    row_deg = rowp.reshape(NW, NCHD, CHD)
    row_agg = rowp.reshape(NW, NCHA, CHA)
    col_agg = colp.reshape(NW, EPWP)
    x_pad = jnp.pad(x, ((0, NP - N_NODES), (0, 0)))

    hist = _deg_kernel(row_deg)
    ybar, dinv = _scale_call(hist[0].reshape(NP, 1), hist[1].reshape(NP, 1),
                             x_pad)
    spart = _agg_kernel(ybar, col_agg, row_agg)
    outp = _final_call(spart[0], spart[1], dinv, W)
    return outp[:N_NODES]
